# baseline (speedup 1.0000x reference)
import functools

import jax
import jax.numpy as jnp
from jax import lax
from jax.experimental import pallas as pl
from jax.experimental.pallas import tpu as pltpu
from jax.experimental.pallas import tpu_sc as plsc

VOCAB = 32000
D = 4096
B_TOT = 4 * 8192
NC, NS = 2, 16
K = 8
NSB = 15
NA = 98
NB_CH = 32
RX = (NA + NB_CH) * K
RY = NA * K
NLOOP = 16


def _gather_kernel(ids_hbm, table_hbm, out_hbm, idx_v, rowsA0, rowsA1,
                   shared, gA0, gA1, oA0, oA1, gB0, gB1, oB0, oB1):
    rowsA = (rowsA0, rowsA1)
    gA = (gA0, gA1)
    oA = (oA0, oA1)
    gB = (gB0, gB1)
    oB = (oB0, oB1)
    cid = lax.axis_index("c")
    sid = lax.axis_index("s")
    is_x = sid < NSB
    base = jnp.where(is_x, (sid * NC + cid) * RX,
                     2 * NSB * RX + cid * RY)

    @pl.when(is_x)
    def _():
        pltpu.sync_copy(ids_hbm.at[pl.ds(base, RX)], idx_v.at[pl.ds(0, RX)])

    @pl.when(jnp.logical_not(is_x))
    def _():
        pltpu.sync_copy(ids_hbm.at[pl.ds(base, RY)], idx_v.at[pl.ds(0, RY)])

    def rowsB(v):
        return shared.at[sid, v]

    def gatherA(a, u):
        start = pl.multiple_of(a * K, K)
        pltpu.async_copy(
            table_hbm.at[idx_v.at[pl.ds(start, K)]], rowsA[u], gA[u])

    def putA(a, u):
        start = pl.multiple_of(a * K, K)
        pltpu.async_copy(rowsA[u], out_hbm.at[pl.ds(base + start, K)], oA[u])

    def gatherB(bb, v):
        start = pl.multiple_of(RY + bb * K, K)
        vec = idx_v[pl.ds(start, 16)]
        for r in range(K):
            row_id = vec[r]
            pltpu.async_copy(
                table_hbm.at[pl.ds(row_id, 1)],
                rowsB(v).at[pl.ds(r, 1)], gB[v])

    def putB(bb, v):
        start = pl.multiple_of(RY + bb * K, K)
        pltpu.async_copy(rowsB(v), out_hbm.at[pl.ds(base + start, K)], oB[v])

    def wait(ref, sem):
        pltpu.make_async_copy(ref, out_hbm.at[pl.ds(0, K)], sem).wait()

    for u in range(2):
        gatherA(u, u)

        @pl.when(is_x)
        def _():
            gatherB(u, u)

    def body(j, carry):
        for u in range(6):
            a = j * 6 + u
            wait(rowsA[u % 2], gA[u % 2])
            putA(a, u % 2)

            @pl.when(a + 2 < NA)
            def _():
                wait(rowsA[u % 2], oA[u % 2])
                gatherA(a + 2, u % 2)

        for v in range(2):
            bb = j * 2 + v

            @pl.when(is_x)
            def _():
                wait(rowsB(v), gB[v])
                putB(bb, v)

            @pl.when(is_x & (bb + 2 < NB_CH))
            def _():
                wait(rowsB(v), oB[v])
                gatherB(bb + 2, v)

        return carry

    lax.fori_loop(0, NLOOP, body, 0)

    for a in range(NLOOP * 6, NA):
        u = a % 2
        wait(rowsA[u], gA[u])
        putA(a, u)

    for u in range(2):
        wait(rowsA[u], oA[u])

        @pl.when(is_x)
        def _():
            wait(rowsB(u), oB[u])


def kernel(text_ids, text_embedding_weight):
    ids = text_ids.reshape(-1).astype(jnp.int32)
    mesh = plsc.VectorSubcoreMesh(core_axis_name="c", subcore_axis_name="s")
    run = functools.partial(
        pl.kernel,
        mesh=mesh,
        out_type=jax.ShapeDtypeStruct((B_TOT, D), jnp.float32),
        scratch_types=[
            pltpu.VMEM((RX + 16,), jnp.int32),
            pltpu.VMEM((K, D), jnp.float32),
            pltpu.VMEM((K, D), jnp.float32),
            pltpu.VMEM_SHARED((NSB, 2, K, D), jnp.float32),
            pltpu.SemaphoreType.DMA,
            pltpu.SemaphoreType.DMA,
            pltpu.SemaphoreType.DMA,
            pltpu.SemaphoreType.DMA,
            pltpu.SemaphoreType.DMA,
            pltpu.SemaphoreType.DMA,
            pltpu.SemaphoreType.DMA,
            pltpu.SemaphoreType.DMA,
        ],
    )(_gather_kernel)
    out = run(ids, text_embedding_weight)
    return out.reshape(text_ids.shape + (D,))
